# Initial kernel scaffold; baseline (speedup 1.0000x reference)
import functools

import jax
import jax.numpy as jnp
from jax import lax
from jax.experimental import pallas as pl
from jax.experimental.pallas import tpu as pltpu
from jax.experimental.pallas import tpu_sc as plsc

N_NODES = 10000
N_EDGES = 320000
N_GRAPHS = 500
BN_EPS = 1e-5
BN_SCALE = (1.0 + BN_EPS) ** -0.5

NC = 2
NS = 16
NW = NC * NS
CH = 128

K_EDGE = 80
N_ACC = 10016
K_POOL = 3
G_ACC = 512


def _make_edge_scatter(n_tab, d, k_chunks, n_acc, n_out):
  mesh = plsc.VectorSubcoreMesh(core_axis_name="c", subcore_axis_name="s")
  z_per = n_acc // NS
  o_per = n_out // NS

  def body(table_hbm, src_hbm, dst_hbm, zeros_hbm, out_hbm,
           src_v, dst_v, rows_v, acc, sem):
    c = lax.axis_index("c")
    s = lax.axis_index("s")
    wid = s * NC + c

    pltpu.sync_copy(zeros_hbm.at[pl.ds(s * z_per, z_per)],
                    acc.at[pl.ds(s * z_per, z_per)])
    pltpu.sync_copy(src_hbm.at[wid], src_v)
    pltpu.sync_copy(dst_hbm.at[wid], dst_v)
    plsc.subcore_barrier()

    def step(k, carry):
      pltpu.async_copy(table_hbm.at[src_v.at[k]], rows_v, sem).wait()
      pltpu.sync_copy(rows_v, acc.at[dst_v.at[k]], add=True)
      return carry

    lax.fori_loop(0, k_chunks, step, 0)
    plsc.subcore_barrier()

    pltpu.sync_copy(acc.at[pl.ds(s * o_per, o_per)],
                    out_hbm.at[c, pl.ds(s * o_per, o_per)])

  return pl.kernel(
      body,
      out_type=jax.ShapeDtypeStruct((NC, n_out, d), jnp.float32),
      mesh=mesh,
      scratch_types=[
          pltpu.VMEM((k_chunks, CH), jnp.int32),
          pltpu.VMEM((k_chunks, CH), jnp.int32),
          pltpu.VMEM((CH, d), jnp.float32),
          pltpu.VMEM_SHARED((n_acc, d), jnp.float32),
          pltpu.SemaphoreType.DMA,
      ],
  )


def _dinv_from_deg(deg_ref):
  cnt = deg_ref[0, :, 0:1] + deg_ref[1, :, 0:1]
  return lax.rsqrt(cnt + 1.0)


def _tc_a(x_ref, w_ref, deg_ref, g1_ref):
  dinv = _dinv_from_deg(deg_ref)
  t = jnp.dot(x_ref[...], w_ref[...], preferred_element_type=jnp.float32)
  g1_ref[...] = t * dinv


def _tc_b(s1_ref, g1_ref, deg_ref, b1_ref, ga1_ref, be1_ref, g2_ref):
  dinv = _dinv_from_deg(deg_ref)
  u = dinv * (s1_ref[0] + s1_ref[1] + g1_ref[...]) + b1_ref[...]
  t = u * (BN_SCALE * ga1_ref[...]) + be1_ref[...]
  h1 = jnp.where(t >= 0, t, 0.02 * t)
  g2_ref[...] = h1 * dinv


def _tc_c(s2_ref, g2_ref, deg_ref, w2_ref, b2_ref, ga2_ref, be2_ref,
          h2_ref, g3_ref):
  dinv = _dinv_from_deg(deg_ref)
  v = dinv * (s2_ref[0] + s2_ref[1] + g2_ref[...])
  h2 = (jnp.dot(v, w2_ref[...], preferred_element_type=jnp.float32)
        + b2_ref[...]) * (BN_SCALE * ga2_ref[...]) + be2_ref[...]
  h2_ref[...] = h2
  g3_ref[...] = h2 * dinv


def _tc_d(s3_ref, g3_ref, deg_ref, w3_ref, b3_ref, psum_ref, binds_ref,
          rep_ref, emb_ref):
  dinv = _dinv_from_deg(deg_ref)
  w = dinv * (s3_ref[0] + s3_ref[1] + g3_ref[...])
  emb_ref[...] = jnp.dot(w, w3_ref[...],
                         preferred_element_type=jnp.float32) + b3_ref[...]
  gids = lax.broadcasted_iota(jnp.int32, (1, N_GRAPHS), 1)
  cnts = jnp.sum((binds_ref[...] == gids).astype(jnp.float32), axis=0)
  ps = psum_ref[0, :N_GRAPHS, :] + psum_ref[1, :N_GRAPHS, :]
  rep_ref[...] = ps / jnp.maximum(cnts, 1.0)[:, None]


def _pad_chunks(a, k_chunks, fill):
  total = NW * k_chunks * CH
  a = jnp.concatenate(
      [a, jnp.full((total - a.shape[0],), fill, dtype=jnp.int32)])
  return a.reshape(NW, k_chunks, CH)


@jax.jit
def kernel(x, edge_index, binds, W1, b1, g1, be1, W2, b2, g2, be2, W3, b3):
  f32 = jnp.float32
  src = _pad_chunks(edge_index[0].astype(jnp.int32), K_EDGE, 0)
  dst = _pad_chunks(edge_index[1].astype(jnp.int32), K_EDGE, N_NODES)
  psrc = _pad_chunks(jnp.arange(N_NODES, dtype=jnp.int32), K_POOL, 0)
  pdst = _pad_chunks(binds.astype(jnp.int32), K_POOL, N_GRAPHS)

  ones8 = jnp.ones((8, 8), f32)
  zeros8 = jnp.zeros((N_ACC, 8), f32)
  zeros64 = jnp.zeros((N_ACC, 64), f32)
  zeros128 = jnp.zeros((N_ACC, 128), f32)
  zerosg = jnp.zeros((G_ACC, 128), f32)

  deg_k = _make_edge_scatter(8, 8, K_EDGE, N_ACC, N_NODES)
  scat64 = _make_edge_scatter(N_NODES, 64, K_EDGE, N_ACC, N_NODES)
  scat128 = _make_edge_scatter(N_NODES, 128, K_EDGE, N_ACC, N_NODES)
  pool_k = _make_edge_scatter(N_NODES, 128, K_POOL, G_ACC, G_ACC)

  deg = deg_k(ones8, jnp.zeros_like(src), dst, zeros8)

  g1v = pl.pallas_call(
      _tc_a, out_shape=jax.ShapeDtypeStruct((N_NODES, 64), f32),
  )(x, W1, deg)

  S1 = scat64(g1v, src, dst, zeros64)

  g2v = pl.pallas_call(
      _tc_b, out_shape=jax.ShapeDtypeStruct((N_NODES, 64), f32),
  )(S1, g1v, deg, b1.reshape(1, 64), g1.reshape(1, 64), be1.reshape(1, 64))

  S2 = scat64(g2v, src, dst, zeros64)

  h2, g3v = pl.pallas_call(
      _tc_c, out_shape=(jax.ShapeDtypeStruct((N_NODES, 128), f32),
                        jax.ShapeDtypeStruct((N_NODES, 128), f32)),
  )(S2, g2v, deg, W2, b2.reshape(1, 128), g2.reshape(1, 128),
    be2.reshape(1, 128))

  S3 = scat128(g3v, src, dst, zeros128)
  psum = pool_k(h2, psrc, pdst, zerosg)

  x_rep, x_emb = pl.pallas_call(
      _tc_d, out_shape=(jax.ShapeDtypeStruct((N_GRAPHS, 128), f32),
                        jax.ShapeDtypeStruct((N_NODES, 256), f32)),
  )(S3, g3v, deg, W3, b3.reshape(1, 256), psum,
    binds.astype(jnp.int32).reshape(N_NODES, 1))

  return (x_rep, x_emb)


# speedup vs baseline: 4.6718x; 4.6718x over previous
import functools

import jax
import jax.numpy as jnp
from jax import lax
from jax.experimental import pallas as pl
from jax.experimental.pallas import tpu as pltpu
from jax.experimental.pallas import tpu_sc as plsc

N_NODES = 10000
N_EDGES = 320000
N_GRAPHS = 500
BN_EPS = 1e-5
BN_SCALE = (1.0 + BN_EPS) ** -0.5

NC = 2
NS = 16
NW = NC * NS
CH = 128

K_EDGE = 80
N_ACC = 10112
K_POOL = 3
G_ACC = 512


def _make_edge_scatter(n_tab, d, k_chunks, n_acc, n_out):
  mesh = plsc.VectorSubcoreMesh(core_axis_name="c", subcore_axis_name="s")
  z_per = n_acc // NS
  o_per = n_out // NS

  def body(table_hbm, src_hbm, dst_hbm, zeros_hbm, out_hbm,
           src_v, dst_v, rows_v, acc, sem):
    c = lax.axis_index("c")
    s = lax.axis_index("s")
    wid = s * NC + c

    pltpu.sync_copy(zeros_hbm.at[pl.ds(s * z_per, z_per)],
                    acc.at[pl.ds(s * z_per, z_per)])
    pltpu.sync_copy(src_hbm.at[wid], src_v)
    pltpu.sync_copy(dst_hbm.at[wid], dst_v)
    plsc.subcore_barrier()

    def step(k, carry):
      pltpu.async_copy(table_hbm.at[src_v.at[k]], rows_v, sem).wait()
      pltpu.sync_copy(rows_v, acc.at[dst_v.at[k]], add=True)
      return carry

    lax.fori_loop(0, k_chunks, step, 0)
    plsc.subcore_barrier()

    pltpu.sync_copy(acc.at[pl.ds(s * o_per, o_per)],
                    out_hbm.at[c, pl.ds(s * o_per, o_per)])

  return pl.kernel(
      body,
      out_type=jax.ShapeDtypeStruct((NC, n_out, d), jnp.float32),
      mesh=mesh,
      compiler_params=pltpu.CompilerParams(use_tc_tiling_on_sc=False),
      scratch_types=[
          pltpu.VMEM((k_chunks, CH), jnp.int32),
          pltpu.VMEM((k_chunks, CH), jnp.int32),
          pltpu.VMEM((CH, d), jnp.float32),
          pltpu.VMEM_SHARED((n_acc, d), jnp.float32),
          pltpu.SemaphoreType.DMA,
      ],
  )


def _dinv_from_deg(deg_ref):
  cnt = deg_ref[0, :N_NODES, 0:1] + deg_ref[1, :N_NODES, 0:1]
  return lax.rsqrt(cnt + 1.0)


def _tc_a(x_ref, w_ref, deg_ref, g1_ref):
  dinv = _dinv_from_deg(deg_ref)
  t = jnp.dot(x_ref[...], w_ref[...], preferred_element_type=jnp.float32)
  g1_ref[...] = t * dinv


def _tc_b(s1_ref, g1_ref, deg_ref, b1_ref, ga1_ref, be1_ref, g2_ref):
  dinv = _dinv_from_deg(deg_ref)
  u = dinv * (s1_ref[0, :N_NODES, :] + s1_ref[1, :N_NODES, :] + g1_ref[...]) + b1_ref[...]
  t = u * (BN_SCALE * ga1_ref[...]) + be1_ref[...]
  h1 = jnp.where(t >= 0, t, 0.02 * t)
  g2_ref[...] = h1 * dinv


def _tc_c(s2_ref, g2_ref, deg_ref, w2_ref, b2_ref, ga2_ref, be2_ref,
          h2_ref, g3_ref):
  dinv = _dinv_from_deg(deg_ref)
  v = dinv * (s2_ref[0, :N_NODES, :] + s2_ref[1, :N_NODES, :] + g2_ref[...])
  h2 = (jnp.dot(v, w2_ref[...], preferred_element_type=jnp.float32)
        + b2_ref[...]) * (BN_SCALE * ga2_ref[...]) + be2_ref[...]
  h2_ref[...] = h2
  g3_ref[...] = h2 * dinv


def _tc_d(s3_ref, g3_ref, deg_ref, w3_ref, b3_ref, psum_ref, binds_ref,
          rep_ref, emb_ref):
  dinv = _dinv_from_deg(deg_ref)
  w = dinv * (s3_ref[0, :N_NODES, :] + s3_ref[1, :N_NODES, :] + g3_ref[...])
  emb_ref[...] = jnp.dot(w, w3_ref[...],
                         preferred_element_type=jnp.float32) + b3_ref[...]
  gids = lax.broadcasted_iota(jnp.int32, (1, N_GRAPHS), 1)
  cnts = jnp.sum((binds_ref[...] == gids).astype(jnp.float32), axis=0)
  ps = psum_ref[0, :N_GRAPHS, :] + psum_ref[1, :N_GRAPHS, :]
  rep_ref[...] = ps / jnp.maximum(cnts, 1.0)[:, None]


def _pad_chunks(a, k_chunks, fill):
  total = NW * k_chunks * CH
  a = jnp.concatenate(
      [a, jnp.full((total - a.shape[0],), fill, dtype=jnp.int32)])
  return a.reshape(NW, k_chunks, CH)


@jax.jit
def kernel(x, edge_index, binds, W1, b1, g1, be1, W2, b2, g2, be2, W3, b3):
  f32 = jnp.float32
  src = _pad_chunks(edge_index[0].astype(jnp.int32), K_EDGE, 0)
  dst = _pad_chunks(edge_index[1].astype(jnp.int32), K_EDGE, N_NODES)
  psrc = _pad_chunks(jnp.arange(N_NODES, dtype=jnp.int32), K_POOL, 0)
  pdst = _pad_chunks(binds.astype(jnp.int32), K_POOL, N_GRAPHS)

  ones8 = jnp.ones((8, 8), f32)
  zeros8 = jnp.zeros((N_ACC, 8), f32)
  zeros64 = jnp.zeros((N_ACC, 64), f32)
  zeros128 = jnp.zeros((N_ACC, 128), f32)
  zerosg = jnp.zeros((G_ACC, 128), f32)

  deg_k = _make_edge_scatter(8, 8, K_EDGE, N_ACC, N_ACC)
  scat64 = _make_edge_scatter(N_NODES, 64, K_EDGE, N_ACC, N_ACC)
  scat128 = _make_edge_scatter(N_NODES, 128, K_EDGE, N_ACC, N_ACC)
  pool_k = _make_edge_scatter(N_NODES, 128, K_POOL, G_ACC, G_ACC)

  deg = deg_k(ones8, jnp.zeros_like(src), dst, zeros8)

  g1v = pl.pallas_call(
      _tc_a, out_shape=jax.ShapeDtypeStruct((N_NODES, 64), f32),
  )(x, W1, deg)

  S1 = scat64(g1v, src, dst, zeros64)

  g2v = pl.pallas_call(
      _tc_b, out_shape=jax.ShapeDtypeStruct((N_NODES, 64), f32),
  )(S1, g1v, deg, b1.reshape(1, 64), g1.reshape(1, 64), be1.reshape(1, 64))

  S2 = scat64(g2v, src, dst, zeros64)

  h2, g3v = pl.pallas_call(
      _tc_c, out_shape=(jax.ShapeDtypeStruct((N_NODES, 128), f32),
                        jax.ShapeDtypeStruct((N_NODES, 128), f32)),
  )(S2, g2v, deg, W2, b2.reshape(1, 128), g2.reshape(1, 128),
    be2.reshape(1, 128))

  S3 = scat128(g3v, src, dst, zeros128)
  psum = pool_k(h2, psrc, pdst, zerosg)

  x_rep, x_emb = pl.pallas_call(
      _tc_d, out_shape=(jax.ShapeDtypeStruct((N_GRAPHS, 128), f32),
                        jax.ShapeDtypeStruct((N_NODES, 256), f32)),
  )(S3, g3v, deg, W3, b3.reshape(1, 256), psum,
    binds.astype(jnp.int32).reshape(N_NODES, 1))

  return (x_rep, x_emb)


# speedup vs baseline: 9.5278x; 2.0394x over previous
import functools

import jax
import jax.numpy as jnp
from jax import lax
from jax.experimental import pallas as pl
from jax.experimental.pallas import tpu as pltpu
from jax.experimental.pallas import tpu_sc as plsc

N_NODES = 10000
N_EDGES = 320000
N_GRAPHS = 500
BN_EPS = 1e-5
BN_SCALE = (1.0 + BN_EPS) ** -0.5

NC = 2
NS = 16
NW = NC * NS
CH = 128

K_EDGE = 80
N_ACC = 10112
K_POOL = 4
G_ACC = 512


def _make_edge_scatter(n_tab, d, k_chunks, n_acc, n_out):
  mesh = plsc.VectorSubcoreMesh(core_axis_name="c", subcore_axis_name="s")
  z_per = n_acc // NS
  o_per = n_out // NS

  assert k_chunks % 2 == 0

  def body(table_hbm, src_hbm, dst_hbm, zeros_hbm, out_hbm,
           src_v, dst_v, rows0, rows1, acc, gsem0, gsem1):
    c = lax.axis_index("c")
    s = lax.axis_index("s")
    wid = s * NC + c

    pltpu.sync_copy(zeros_hbm.at[pl.ds(s * z_per, z_per)],
                    acc.at[pl.ds(s * z_per, z_per)])
    pltpu.sync_copy(src_hbm.at[wid], src_v)
    pltpu.sync_copy(dst_hbm.at[wid], dst_v)
    plsc.subcore_barrier()

    pltpu.async_copy(table_hbm.at[src_v.at[0]], rows0, gsem0)

    def step(k2, carry):
      k = 2 * k2
      pltpu.make_async_copy(table_hbm.at[src_v.at[k]], rows0, gsem0).wait()
      pltpu.async_copy(table_hbm.at[src_v.at[k + 1]], rows1, gsem1)
      pltpu.sync_copy(rows0, acc.at[dst_v.at[k]], add=True)
      pltpu.make_async_copy(
          table_hbm.at[src_v.at[k + 1]], rows1, gsem1).wait()

      @pl.when(k + 2 < k_chunks)
      def _():
        pltpu.async_copy(table_hbm.at[src_v.at[k + 2]], rows0, gsem0)

      pltpu.sync_copy(rows1, acc.at[dst_v.at[k + 1]], add=True)
      return carry

    lax.fori_loop(0, k_chunks // 2, step, 0)
    plsc.subcore_barrier()

    pltpu.sync_copy(acc.at[pl.ds(s * o_per, o_per)],
                    out_hbm.at[c, pl.ds(s * o_per, o_per)])

  return pl.kernel(
      body,
      out_type=jax.ShapeDtypeStruct((NC, n_out, d), jnp.float32),
      mesh=mesh,
      compiler_params=pltpu.CompilerParams(use_tc_tiling_on_sc=False),
      scratch_types=[
          pltpu.VMEM((k_chunks, CH), jnp.int32),
          pltpu.VMEM((k_chunks, CH), jnp.int32),
          pltpu.VMEM((CH, d), jnp.float32),
          pltpu.VMEM((CH, d), jnp.float32),
          pltpu.VMEM_SHARED((n_acc, d), jnp.float32),
          pltpu.SemaphoreType.DMA,
          pltpu.SemaphoreType.DMA,
      ],
  )


def _make_hist(d, k_chunks, n_acc, n_out):
  mesh = plsc.VectorSubcoreMesh(core_axis_name="c", subcore_axis_name="s")
  z_per = n_acc // NS
  o_per = n_out // NS
  assert k_chunks % 2 == 0

  def body(ones_hbm, dst_hbm, zeros_hbm, out_hbm,
           dst_v, rows_v, acc, sem0, sem1):
    c = lax.axis_index("c")
    s = lax.axis_index("s")
    wid = s * NC + c

    pltpu.sync_copy(zeros_hbm.at[pl.ds(s * z_per, z_per)],
                    acc.at[pl.ds(s * z_per, z_per)])
    pltpu.sync_copy(dst_hbm.at[wid], dst_v)
    pltpu.sync_copy(ones_hbm, rows_v)
    plsc.subcore_barrier()

    pltpu.async_copy(rows_v, acc.at[dst_v.at[0]], sem0, add=True)

    def step(k2, carry):
      k = 2 * k2
      pltpu.async_copy(rows_v, acc.at[dst_v.at[k + 1]], sem1, add=True)
      pltpu.make_async_copy(rows_v, acc.at[dst_v.at[k]], sem0).wait()

      @pl.when(k + 2 < k_chunks)
      def _():
        pltpu.async_copy(rows_v, acc.at[dst_v.at[k + 2]], sem0, add=True)

      pltpu.make_async_copy(rows_v, acc.at[dst_v.at[k + 1]], sem1).wait()
      return carry

    lax.fori_loop(0, k_chunks // 2, step, 0)
    plsc.subcore_barrier()
    pltpu.sync_copy(acc.at[pl.ds(s * o_per, o_per)],
                    out_hbm.at[c, pl.ds(s * o_per, o_per)])

  return pl.kernel(
      body,
      out_type=jax.ShapeDtypeStruct((NC, n_out, d), jnp.float32),
      mesh=mesh,
      compiler_params=pltpu.CompilerParams(use_tc_tiling_on_sc=False),
      scratch_types=[
          pltpu.VMEM((k_chunks, CH), jnp.int32),
          pltpu.VMEM((CH, d), jnp.float32),
          pltpu.VMEM_SHARED((n_acc, d), jnp.float32),
          pltpu.SemaphoreType.DMA,
          pltpu.SemaphoreType.DMA,
      ],
  )


def _dinv_from_deg(deg_ref):
  cnt = deg_ref[0, :N_NODES, 0:1] + deg_ref[1, :N_NODES, 0:1]
  return lax.rsqrt(cnt + 1.0)


def _tc_a(x_ref, w_ref, deg_ref, g1_ref):
  dinv = _dinv_from_deg(deg_ref)
  t = jnp.dot(x_ref[...], w_ref[...], preferred_element_type=jnp.float32)
  g1_ref[...] = t * dinv


def _tc_b(s1_ref, g1_ref, deg_ref, b1_ref, ga1_ref, be1_ref, g2_ref):
  dinv = _dinv_from_deg(deg_ref)
  u = dinv * (s1_ref[0, :N_NODES, :] + s1_ref[1, :N_NODES, :] + g1_ref[...]) + b1_ref[...]
  t = u * (BN_SCALE * ga1_ref[...]) + be1_ref[...]
  h1 = jnp.where(t >= 0, t, 0.02 * t)
  g2_ref[...] = h1 * dinv


def _tc_c(s2_ref, g2_ref, deg_ref, w2_ref, b2_ref, ga2_ref, be2_ref,
          h2_ref, g3a_ref, g3b_ref):
  dinv = _dinv_from_deg(deg_ref)
  v = dinv * (s2_ref[0, :N_NODES, :] + s2_ref[1, :N_NODES, :] + g2_ref[...])
  h2 = (jnp.dot(v, w2_ref[...], preferred_element_type=jnp.float32)
        + b2_ref[...]) * (BN_SCALE * ga2_ref[...]) + be2_ref[...]
  h2_ref[...] = h2
  g3 = h2 * dinv
  g3a_ref[...] = g3[:, :64]
  g3b_ref[...] = g3[:, 64:]


def _tc_d(s3a_ref, s3b_ref, g3a_ref, g3b_ref, deg_ref, w3_ref, b3_ref,
          emb_ref):
  cnt = deg_ref[0, :, 0:1] + deg_ref[1, :, 0:1]
  dinv = lax.rsqrt(cnt + 1.0)
  wa = s3a_ref[0] + s3a_ref[1] + g3a_ref[...]
  wb = s3b_ref[0] + s3b_ref[1] + g3b_ref[...]
  w = dinv * jnp.concatenate([wa, wb], axis=1)
  emb_ref[...] = jnp.dot(w, w3_ref[...],
                         preferred_element_type=jnp.float32) + b3_ref[...]


def _tc_e(psum_ref, pcnt_ref, rep_ref):
  cnt = pcnt_ref[0, :N_GRAPHS, 0:1] + pcnt_ref[1, :N_GRAPHS, 0:1]
  ps = psum_ref[0, :N_GRAPHS, :] + psum_ref[1, :N_GRAPHS, :]
  rep_ref[...] = ps / jnp.maximum(cnt, 1.0)


def _pad_chunks(a, k_chunks, fill):
  total = NW * k_chunks * CH
  a = jnp.concatenate(
      [a, jnp.full((total - a.shape[0],), fill, dtype=jnp.int32)])
  return a.reshape(NW, k_chunks, CH)


@jax.jit
def kernel(x, edge_index, binds, W1, b1, g1, be1, W2, b2, g2, be2, W3, b3):
  f32 = jnp.float32
  src = _pad_chunks(edge_index[0].astype(jnp.int32), K_EDGE, 0)
  dst = _pad_chunks(edge_index[1].astype(jnp.int32), K_EDGE, N_NODES)
  psrc = _pad_chunks(jnp.arange(N_NODES, dtype=jnp.int32), K_POOL, 0)
  pdst = _pad_chunks(binds.astype(jnp.int32), K_POOL, N_GRAPHS)

  ones8 = jnp.ones((CH, 8), f32)
  zeros8 = jnp.zeros((N_ACC, 8), f32)
  zeros64 = jnp.zeros((N_ACC, 64), f32)
  zerosg = jnp.zeros((G_ACC, 128), f32)

  deg_k = _make_hist(8, K_EDGE, N_ACC, N_ACC)
  cnt_k = _make_hist(8, K_POOL, G_ACC, G_ACC)
  scat64 = _make_edge_scatter(N_NODES, 64, K_EDGE, N_ACC, N_ACC)
  pool_k = _make_edge_scatter(N_NODES, 128, K_POOL, G_ACC, G_ACC)

  deg = deg_k(ones8, dst, zeros8)

  g1v = pl.pallas_call(
      _tc_a, out_shape=jax.ShapeDtypeStruct((N_NODES, 64), f32),
  )(x, W1, deg)

  S1 = scat64(g1v, src, dst, zeros64)

  g2v = pl.pallas_call(
      _tc_b, out_shape=jax.ShapeDtypeStruct((N_NODES, 64), f32),
  )(S1, g1v, deg, b1.reshape(1, 64), g1.reshape(1, 64), be1.reshape(1, 64))

  S2 = scat64(g2v, src, dst, zeros64)

  h2, g3a, g3b = pl.pallas_call(
      _tc_c, out_shape=(jax.ShapeDtypeStruct((N_NODES, 128), f32),
                        jax.ShapeDtypeStruct((N_NODES, 64), f32),
                        jax.ShapeDtypeStruct((N_NODES, 64), f32)),
  )(S2, g2v, deg, W2, b2.reshape(1, 128), g2.reshape(1, 128),
    be2.reshape(1, 128))

  S3a = scat64(g3a, src, dst, zeros64)
  S3b = scat64(g3b, src, dst, zeros64)
  psum = pool_k(h2, psrc, pdst, zerosg)
  pcnt = cnt_k(jnp.ones((CH, 8), f32), pdst, jnp.zeros((G_ACC, 8), f32))

  B = 2000
  x_emb = pl.pallas_call(
      _tc_d,
      grid=(N_NODES // B,),
      in_specs=[
          pl.BlockSpec((2, B, 64), lambda i: (0, i, 0)),
          pl.BlockSpec((2, B, 64), lambda i: (0, i, 0)),
          pl.BlockSpec((B, 64), lambda i: (i, 0)),
          pl.BlockSpec((B, 64), lambda i: (i, 0)),
          pl.BlockSpec((2, B, 8), lambda i: (0, i, 0)),
          pl.BlockSpec((128, 256), lambda i: (0, 0)),
          pl.BlockSpec((1, 256), lambda i: (0, 0)),
      ],
      out_specs=pl.BlockSpec((B, 256), lambda i: (i, 0)),
      out_shape=jax.ShapeDtypeStruct((N_NODES, 256), f32),
  )(S3a, S3b, g3a, g3b, deg, W3, b3.reshape(1, 256))

  x_rep = pl.pallas_call(
      _tc_e, out_shape=jax.ShapeDtypeStruct((N_GRAPHS, 128), f32),
  )(psum, pcnt)

  return (x_rep, x_emb)


# speedup vs baseline: 10.9222x; 1.1463x over previous
import jax
import jax.numpy as jnp
from jax import lax
from jax.experimental import pallas as pl
from jax.experimental.pallas import tpu as pltpu
from jax.experimental.pallas import tpu_sc as plsc

N_NODES = 10000
N_GRAPHS = 500
BN_EPS = 1e-5
BN_SCALE = (1.0 + BN_EPS) ** -0.5

NC = 2
NS = 16
NW = NC * NS
CH = 128

K_EDGE = 80
N_ACC = 10112


def _make_edge_scatter(d, k_chunks, n_acc):
  mesh = plsc.VectorSubcoreMesh(core_axis_name="c", subcore_axis_name="s")
  z_per = n_acc // NS
  assert k_chunks % 4 == 0 and z_per % 8 == 0

  def body(table_hbm, src_hbm, dst_hbm, zeros_hbm, out_hbm,
           src_v, dst_v, b0, b1, b2, b3, acc,
           g0, g1, g2, g3, s0, s1, s2, s3):
    c = lax.axis_index("c")
    s = lax.axis_index("s")
    wid = s * NC + c
    bufs = (b0, b1, b2, b3)
    gsems = (g0, g1, g2, g3)
    ssems = (s0, s1, s2, s3)

    pltpu.sync_copy(zeros_hbm.at[pl.ds(s * z_per, z_per)],
                    acc.at[pl.ds(s * z_per, z_per)])
    pltpu.sync_copy(src_hbm.at[wid], src_v)
    pltpu.sync_copy(dst_hbm.at[wid], dst_v)
    plsc.subcore_barrier()

    def gather(k, j):
      pltpu.async_copy(table_hbm.at[src_v.at[k]], bufs[j], gsems[j])

    def scatter(k, j):
      pltpu.async_copy(bufs[j], acc.at[dst_v.at[k]], ssems[j], add=True)

    def wait_gather(k, j):
      pltpu.make_async_copy(table_hbm.at[src_v.at[k]], bufs[j],
                            gsems[j]).wait()

    def wait_scatter(k, j):
      pltpu.make_async_copy(bufs[j], acc.at[dst_v.at[k]], ssems[j]).wait()

    gather(0, 0)

    def step(k2, carry):
      k = 2 * k2
      wait_gather(k, 0)
      gather(k + 1, 1)
      pltpu.sync_copy(bufs[0], acc.at[dst_v.at[k]], add=True)
      wait_gather(k + 1, 1)

      @pl.when(k + 2 < k_chunks)
      def _():
        gather(k + 2, 0)

      pltpu.sync_copy(bufs[1], acc.at[dst_v.at[k + 1]], add=True)
      return carry

    lax.fori_loop(0, k_chunks // 2, step, 0)
    plsc.subcore_barrier()

    pltpu.sync_copy(acc.at[pl.ds(s * z_per, z_per)],
                    out_hbm.at[c, pl.ds(s * z_per, z_per)])

  return pl.kernel(
      body,
      out_type=jax.ShapeDtypeStruct((NC, n_acc, d), jnp.float32),
      mesh=mesh,
      compiler_params=pltpu.CompilerParams(use_tc_tiling_on_sc=False),
      scratch_types=[
          pltpu.VMEM((k_chunks, CH), jnp.int32),
          pltpu.VMEM((k_chunks, CH), jnp.int32),
          pltpu.VMEM((CH, d), jnp.float32),
          pltpu.VMEM((CH, d), jnp.float32),
          pltpu.VMEM((CH, d), jnp.float32),
          pltpu.VMEM((CH, d), jnp.float32),
          pltpu.VMEM_SHARED((n_acc, d), jnp.float32),
          pltpu.SemaphoreType.DMA,
          pltpu.SemaphoreType.DMA,
          pltpu.SemaphoreType.DMA,
          pltpu.SemaphoreType.DMA,
          pltpu.SemaphoreType.DMA,
          pltpu.SemaphoreType.DMA,
          pltpu.SemaphoreType.DMA,
          pltpu.SemaphoreType.DMA,
      ],
  )


def _make_hist(d, k_chunks, n_acc):
  mesh = plsc.VectorSubcoreMesh(core_axis_name="c", subcore_axis_name="s")
  z_per = n_acc // NS
  assert k_chunks % 4 == 0 and z_per % 8 == 0

  def body(ones_hbm, dst_hbm, zeros_hbm, out_hbm,
           dst_v, rows_v, acc, s0, s1, s2, s3):
    c = lax.axis_index("c")
    s = lax.axis_index("s")
    wid = s * NC + c
    ssems = (s0, s1, s2, s3)

    pltpu.sync_copy(zeros_hbm.at[pl.ds(s * z_per, z_per)],
                    acc.at[pl.ds(s * z_per, z_per)])
    pltpu.sync_copy(dst_hbm.at[wid], dst_v)
    pltpu.sync_copy(ones_hbm, rows_v)
    plsc.subcore_barrier()

    def scatter(k, jj):
      pltpu.async_copy(rows_v, acc.at[dst_v.at[k]], ssems[jj], add=True)

    def wait_scatter(k, jj):
      pltpu.make_async_copy(rows_v, acc.at[dst_v.at[k]], ssems[jj]).wait()

    scatter(0, 0)
    scatter(1, 1)
    scatter(2, 2)

    def step(m, carry):
      base = 4 * m
      for j in range(4):
        k = base + j

        @pl.when(k + 3 < k_chunks)
        def _():
          scatter(k + 3, (j + 3) % 4)

        wait_scatter(k, j)
      return carry

    lax.fori_loop(0, k_chunks // 4, step, 0)
    plsc.subcore_barrier()
    pltpu.sync_copy(acc.at[pl.ds(s * z_per, z_per)],
                    out_hbm.at[c, pl.ds(s * z_per, z_per)])

  return pl.kernel(
      body,
      out_type=jax.ShapeDtypeStruct((NC, n_acc, d), jnp.float32),
      mesh=mesh,
      compiler_params=pltpu.CompilerParams(use_tc_tiling_on_sc=False),
      scratch_types=[
          pltpu.VMEM((k_chunks, CH), jnp.int32),
          pltpu.VMEM((CH, d), jnp.float32),
          pltpu.VMEM_SHARED((n_acc, d), jnp.float32),
          pltpu.SemaphoreType.DMA,
          pltpu.SemaphoreType.DMA,
          pltpu.SemaphoreType.DMA,
          pltpu.SemaphoreType.DMA,
      ],
  )


def _dinv_from_deg(deg_ref):
  cnt = deg_ref[0, :N_NODES, 0:1] + deg_ref[1, :N_NODES, 0:1]
  return lax.rsqrt(cnt + 1.0)


def _tc_a(x_ref, w_ref, deg_ref, g1_ref):
  dinv = _dinv_from_deg(deg_ref)
  t = jnp.dot(x_ref[...], w_ref[...], preferred_element_type=jnp.float32)
  g1_ref[...] = t * dinv


def _tc_b(s1_ref, g1_ref, deg_ref, b1_ref, ga1_ref, be1_ref, g2_ref):
  dinv = _dinv_from_deg(deg_ref)
  u = dinv * (s1_ref[0, :N_NODES, :] + s1_ref[1, :N_NODES, :]
              + g1_ref[...]) + b1_ref[...]
  t = u * (BN_SCALE * ga1_ref[...]) + be1_ref[...]
  h1 = jnp.where(t >= 0, t, 0.02 * t)
  g2_ref[...] = h1 * dinv


def _tc_c(s2_ref, g2_ref, deg_ref, w2_ref, b2_ref, ga2_ref, be2_ref,
          h2_ref, g3a_ref, g3b_ref):
  dinv = _dinv_from_deg(deg_ref)
  v = dinv * (s2_ref[0, :N_NODES, :] + s2_ref[1, :N_NODES, :] + g2_ref[...])
  h2 = (jnp.dot(v, w2_ref[...], preferred_element_type=jnp.float32)
        + b2_ref[...]) * (BN_SCALE * ga2_ref[...]) + be2_ref[...]
  h2_ref[...] = h2
  g3 = h2 * dinv
  g3a_ref[...] = g3[:, :64]
  g3b_ref[...] = g3[:, 64:]


def _tc_d(s3a_ref, s3b_ref, g3a_ref, g3b_ref, deg_ref, w3_ref, b3_ref,
          emb_ref):
  cnt = deg_ref[0, :, 0:1] + deg_ref[1, :, 0:1]
  dinv = lax.rsqrt(cnt + 1.0)
  wa = s3a_ref[0] + s3a_ref[1] + g3a_ref[...]
  wb = s3b_ref[0] + s3b_ref[1] + g3b_ref[...]
  w = dinv * jnp.concatenate([wa, wb], axis=1)
  emb_ref[...] = jnp.dot(w, w3_ref[...],
                         preferred_element_type=jnp.float32) + b3_ref[...]


def _tc_pool(nblk):
  def body(binds_ref, h2_ref, rep_ref, cnt_s):
    i = pl.program_id(0)
    gids = lax.broadcasted_iota(jnp.int32, (1, N_GRAPHS), 1)
    oh = (binds_ref[...] == gids).astype(jnp.float32)
    bs = lax.dot_general(oh, h2_ref[...], (((0,), (0,)), ((), ())),
                         preferred_element_type=jnp.float32)
    bc = jnp.sum(oh, axis=0)[:, None]

    @pl.when(i == 0)
    def _():
      rep_ref[...] = bs
      cnt_s[...] = bc

    @pl.when(i > 0)
    def _():
      rep_ref[...] += bs
      cnt_s[...] += bc

    @pl.when(i == nblk - 1)
    def _():
      rep_ref[...] = rep_ref[...] / jnp.maximum(cnt_s[...], 1.0)

  return body


def _pad_chunks(a, k_chunks, fill):
  total = NW * k_chunks * CH
  a = jnp.concatenate(
      [a, jnp.full((total - a.shape[0],), fill, dtype=jnp.int32)])
  return a.reshape(NW, k_chunks, CH)


@jax.jit
def kernel(x, edge_index, binds, W1, b1, g1, be1, W2, b2, g2, be2, W3, b3):
  f32 = jnp.float32
  src = _pad_chunks(edge_index[0].astype(jnp.int32), K_EDGE, 0)
  dst = _pad_chunks(edge_index[1].astype(jnp.int32), K_EDGE, N_NODES)

  ones_h = jnp.ones((CH, 8), f32)
  zeros_h = jnp.zeros((N_ACC, 8), f32)
  zeros64 = jnp.zeros((N_ACC, 64), f32)

  deg_k = _make_hist(8, K_EDGE, N_ACC)
  scat64 = _make_edge_scatter(64, K_EDGE, N_ACC)

  deg = deg_k(ones_h, dst, zeros_h)

  g1v = pl.pallas_call(
      _tc_a, out_shape=jax.ShapeDtypeStruct((N_NODES, 64), f32),
  )(x, W1, deg)

  S1 = scat64(g1v, src, dst, zeros64)

  g2v = pl.pallas_call(
      _tc_b, out_shape=jax.ShapeDtypeStruct((N_NODES, 64), f32),
  )(S1, g1v, deg, b1.reshape(1, 64), g1.reshape(1, 64), be1.reshape(1, 64))

  S2 = scat64(g2v, src, dst, zeros64)

  h2, g3a, g3b = pl.pallas_call(
      _tc_c, out_shape=(jax.ShapeDtypeStruct((N_NODES, 128), f32),
                        jax.ShapeDtypeStruct((N_NODES, 64), f32),
                        jax.ShapeDtypeStruct((N_NODES, 64), f32)),
  )(S2, g2v, deg, W2, b2.reshape(1, 128), g2.reshape(1, 128),
    be2.reshape(1, 128))

  S3a = scat64(g3a, src, dst, zeros64)
  S3b = scat64(g3b, src, dst, zeros64)

  B = 2000
  x_emb = pl.pallas_call(
      _tc_d,
      grid=(N_NODES // B,),
      in_specs=[
          pl.BlockSpec((2, B, 64), lambda i: (0, i, 0)),
          pl.BlockSpec((2, B, 64), lambda i: (0, i, 0)),
          pl.BlockSpec((B, 64), lambda i: (i, 0)),
          pl.BlockSpec((B, 64), lambda i: (i, 0)),
          pl.BlockSpec((2, B, 8), lambda i: (0, i, 0)),
          pl.BlockSpec((128, 256), lambda i: (0, 0)),
          pl.BlockSpec((1, 256), lambda i: (0, 0)),
      ],
      out_specs=pl.BlockSpec((B, 256), lambda i: (i, 0)),
      out_shape=jax.ShapeDtypeStruct((N_NODES, 256), f32),
  )(S3a, S3b, g3a, g3b, deg, W3, b3.reshape(1, 256))

  PB = 2000
  x_rep = pl.pallas_call(
      _tc_pool(N_NODES // PB),
      grid=(N_NODES // PB,),
      in_specs=[
          pl.BlockSpec((PB, 1), lambda i: (i, 0)),
          pl.BlockSpec((PB, 128), lambda i: (i, 0)),
      ],
      out_specs=pl.BlockSpec((N_GRAPHS, 128), lambda i: (0, 0)),
      out_shape=jax.ShapeDtypeStruct((N_GRAPHS, 128), f32),
      scratch_shapes=[pltpu.VMEM((N_GRAPHS, 1), f32)],
  )(binds.astype(jnp.int32).reshape(N_NODES, 1), h2)

  return (x_rep, x_emb)


# speedup vs baseline: 11.7610x; 1.0768x over previous
import jax
import jax.numpy as jnp
from jax import lax
from jax.experimental import pallas as pl
from jax.experimental.pallas import tpu as pltpu
from jax.experimental.pallas import tpu_sc as plsc

N_NODES = 10000
N_GRAPHS = 500
BN_EPS = 1e-5
BN_SCALE = (1.0 + BN_EPS) ** -0.5

NC = 2
NS = 16
NW = NC * NS
CH = 128

K_EDGE = 80
N_ACC = 10112


def _make_edge_scatter(d, k_chunks, n_acc):
  mesh = plsc.VectorSubcoreMesh(core_axis_name="c", subcore_axis_name="s")
  z_per = n_acc // NS
  assert k_chunks % 4 == 0 and z_per % 8 == 0

  def body(table_hbm, src_hbm, dst_hbm, zeros_hbm, out_hbm,
           src_v, dst_v, b0, b1, b2, b3, acc,
           g0, g1, g2, g3, s0, s1, s2, s3):
    c = lax.axis_index("c")
    s = lax.axis_index("s")
    wid = s * NC + c
    bufs = (b0, b1, b2, b3)
    gsems = (g0, g1, g2, g3)
    ssems = (s0, s1, s2, s3)

    pltpu.sync_copy(zeros_hbm.at[pl.ds(s * z_per, z_per)],
                    acc.at[pl.ds(s * z_per, z_per)])
    pltpu.sync_copy(src_hbm.at[wid], src_v)
    pltpu.sync_copy(dst_hbm.at[wid], dst_v)
    plsc.subcore_barrier()

    def gather(k, j):
      pltpu.async_copy(table_hbm.at[src_v.at[k]], bufs[j], gsems[j])

    def scatter(k, j):
      pltpu.async_copy(bufs[j], acc.at[dst_v.at[k]], ssems[j], add=True)

    def wait_gather(k, j):
      pltpu.make_async_copy(table_hbm.at[src_v.at[k]], bufs[j],
                            gsems[j]).wait()

    def wait_scatter(k, j):
      pltpu.make_async_copy(bufs[j], acc.at[dst_v.at[k]], ssems[j]).wait()

    gather(0, 0)
    gather(1, 1)

    def step(m, carry):
      base = 4 * m
      for j in range(4):
        k = base + j
        jn = (j + 2) % 4

        @pl.when(k - 2 >= 0)
        def _():
          wait_scatter(k - 2, jn)

        @pl.when(k + 2 < k_chunks)
        def _():
          gather(k + 2, jn)

        wait_gather(k, j)
        scatter(k, j)
      return carry

    lax.fori_loop(0, k_chunks // 4, step, 0)
    wait_scatter(k_chunks - 2, 2)
    wait_scatter(k_chunks - 1, 3)
    plsc.subcore_barrier()

    pltpu.sync_copy(acc.at[pl.ds(s * z_per, z_per)],
                    out_hbm.at[c, pl.ds(s * z_per, z_per)])

  return pl.kernel(
      body,
      out_type=jax.ShapeDtypeStruct((NC, n_acc, d), jnp.float32),
      mesh=mesh,
      compiler_params=pltpu.CompilerParams(use_tc_tiling_on_sc=False),
      scratch_types=[
          pltpu.VMEM((k_chunks, CH), jnp.int32),
          pltpu.VMEM((k_chunks, CH), jnp.int32),
          pltpu.VMEM((CH, d), jnp.float32),
          pltpu.VMEM((CH, d), jnp.float32),
          pltpu.VMEM((CH, d), jnp.float32),
          pltpu.VMEM((CH, d), jnp.float32),
          pltpu.VMEM_SHARED((n_acc, d), jnp.float32),
          pltpu.SemaphoreType.DMA,
          pltpu.SemaphoreType.DMA,
          pltpu.SemaphoreType.DMA,
          pltpu.SemaphoreType.DMA,
          pltpu.SemaphoreType.DMA,
          pltpu.SemaphoreType.DMA,
          pltpu.SemaphoreType.DMA,
          pltpu.SemaphoreType.DMA,
      ],
  )


def _make_hist(d, k_chunks, n_acc):
  mesh = plsc.VectorSubcoreMesh(core_axis_name="c", subcore_axis_name="s")
  z_per = n_acc // NS
  assert k_chunks % 4 == 0 and z_per % 8 == 0

  def body(ones_hbm, dst_hbm, zeros_hbm, out_hbm,
           dst_v, rows_v, acc, s0, s1, s2, s3):
    c = lax.axis_index("c")
    s = lax.axis_index("s")
    wid = s * NC + c
    ssems = (s0, s1, s2, s3)

    pltpu.sync_copy(zeros_hbm.at[pl.ds(s * z_per, z_per)],
                    acc.at[pl.ds(s * z_per, z_per)])
    pltpu.sync_copy(dst_hbm.at[wid], dst_v)
    pltpu.sync_copy(ones_hbm, rows_v)
    plsc.subcore_barrier()

    def scatter(k, jj):
      pltpu.async_copy(rows_v, acc.at[dst_v.at[k]], ssems[jj], add=True)

    def wait_scatter(k, jj):
      pltpu.make_async_copy(rows_v, acc.at[dst_v.at[k]], ssems[jj]).wait()

    scatter(0, 0)
    scatter(1, 1)
    scatter(2, 2)

    def step(m, carry):
      base = 4 * m
      for j in range(4):
        k = base + j

        @pl.when(k + 3 < k_chunks)
        def _():
          scatter(k + 3, (j + 3) % 4)

        wait_scatter(k, j)
      return carry

    lax.fori_loop(0, k_chunks // 4, step, 0)
    plsc.subcore_barrier()
    pltpu.sync_copy(acc.at[pl.ds(s * z_per, z_per)],
                    out_hbm.at[c, pl.ds(s * z_per, z_per)])

  return pl.kernel(
      body,
      out_type=jax.ShapeDtypeStruct((NC, n_acc, d), jnp.float32),
      mesh=mesh,
      compiler_params=pltpu.CompilerParams(use_tc_tiling_on_sc=False),
      scratch_types=[
          pltpu.VMEM((k_chunks, CH), jnp.int32),
          pltpu.VMEM((CH, d), jnp.float32),
          pltpu.VMEM_SHARED((n_acc, d), jnp.float32),
          pltpu.SemaphoreType.DMA,
          pltpu.SemaphoreType.DMA,
          pltpu.SemaphoreType.DMA,
          pltpu.SemaphoreType.DMA,
      ],
  )


def _dinv_from_deg(deg_ref):
  cnt = deg_ref[0, :N_NODES, 0:1] + deg_ref[1, :N_NODES, 0:1]
  return lax.rsqrt(cnt + 1.0)


def _tc_a(x_ref, w_ref, deg_ref, g1_ref):
  dinv = _dinv_from_deg(deg_ref)
  t = jnp.dot(x_ref[...], w_ref[...], preferred_element_type=jnp.float32)
  g1_ref[...] = t * dinv


def _tc_b(s1_ref, g1_ref, deg_ref, b1_ref, ga1_ref, be1_ref, g2_ref):
  dinv = _dinv_from_deg(deg_ref)
  u = dinv * (s1_ref[0, :N_NODES, :] + s1_ref[1, :N_NODES, :]
              + g1_ref[...]) + b1_ref[...]
  t = u * (BN_SCALE * ga1_ref[...]) + be1_ref[...]
  h1 = jnp.where(t >= 0, t, 0.02 * t)
  g2_ref[...] = h1 * dinv


def _tc_c(s2_ref, g2_ref, deg_ref, w2_ref, b2_ref, ga2_ref, be2_ref,
          h2_ref, g3a_ref, g3b_ref):
  dinv = _dinv_from_deg(deg_ref)
  v = dinv * (s2_ref[0, :N_NODES, :] + s2_ref[1, :N_NODES, :] + g2_ref[...])
  h2 = (jnp.dot(v, w2_ref[...], preferred_element_type=jnp.float32)
        + b2_ref[...]) * (BN_SCALE * ga2_ref[...]) + be2_ref[...]
  h2_ref[...] = h2
  g3 = h2 * dinv
  g3a_ref[...] = g3[:, :64]
  g3b_ref[...] = g3[:, 64:]


def _tc_d(s3a_ref, s3b_ref, g3a_ref, g3b_ref, deg_ref, w3_ref, b3_ref,
          emb_ref):
  cnt = deg_ref[0, :, 0:1] + deg_ref[1, :, 0:1]
  dinv = lax.rsqrt(cnt + 1.0)
  wa = s3a_ref[0] + s3a_ref[1] + g3a_ref[...]
  wb = s3b_ref[0] + s3b_ref[1] + g3b_ref[...]
  w = dinv * jnp.concatenate([wa, wb], axis=1)
  emb_ref[...] = jnp.dot(w, w3_ref[...],
                         preferred_element_type=jnp.float32) + b3_ref[...]


def _tc_pool(nblk):
  def body(binds_ref, h2_ref, rep_ref, cnt_s):
    i = pl.program_id(0)
    gids = lax.broadcasted_iota(jnp.int32, (1, N_GRAPHS), 1)
    oh = (binds_ref[...] == gids).astype(jnp.float32)
    bs = lax.dot_general(oh, h2_ref[...], (((0,), (0,)), ((), ())),
                         preferred_element_type=jnp.float32)
    bc = jnp.sum(oh, axis=0)[:, None]

    @pl.when(i == 0)
    def _():
      rep_ref[...] = bs
      cnt_s[...] = bc

    @pl.when(i > 0)
    def _():
      rep_ref[...] += bs
      cnt_s[...] += bc

    @pl.when(i == nblk - 1)
    def _():
      rep_ref[...] = rep_ref[...] / jnp.maximum(cnt_s[...], 1.0)

  return body


def _pad_chunks(a, k_chunks, fill):
  total = NW * k_chunks * CH
  a = jnp.concatenate(
      [a, jnp.full((total - a.shape[0],), fill, dtype=jnp.int32)])
  return a.reshape(NW, k_chunks, CH)


@jax.jit
def kernel(x, edge_index, binds, W1, b1, g1, be1, W2, b2, g2, be2, W3, b3):
  f32 = jnp.float32
  src = _pad_chunks(edge_index[0].astype(jnp.int32), K_EDGE, 0)
  dst = _pad_chunks(edge_index[1].astype(jnp.int32), K_EDGE, N_NODES)

  ones_h = jnp.ones((CH, 8), f32)
  zeros_h = jnp.zeros((N_ACC, 8), f32)
  zeros64 = jnp.zeros((N_ACC, 64), f32)

  deg_k = _make_hist(8, K_EDGE, N_ACC)
  scat64 = _make_edge_scatter(64, K_EDGE, N_ACC)

  deg = deg_k(ones_h, dst, zeros_h)

  g1v = pl.pallas_call(
      _tc_a, out_shape=jax.ShapeDtypeStruct((N_NODES, 64), f32),
  )(x, W1, deg)

  S1 = scat64(g1v, src, dst, zeros64)

  g2v = pl.pallas_call(
      _tc_b, out_shape=jax.ShapeDtypeStruct((N_NODES, 64), f32),
  )(S1, g1v, deg, b1.reshape(1, 64), g1.reshape(1, 64), be1.reshape(1, 64))

  S2 = scat64(g2v, src, dst, zeros64)

  h2, g3a, g3b = pl.pallas_call(
      _tc_c, out_shape=(jax.ShapeDtypeStruct((N_NODES, 128), f32),
                        jax.ShapeDtypeStruct((N_NODES, 64), f32),
                        jax.ShapeDtypeStruct((N_NODES, 64), f32)),
  )(S2, g2v, deg, W2, b2.reshape(1, 128), g2.reshape(1, 128),
    be2.reshape(1, 128))

  S3a = scat64(g3a, src, dst, zeros64)
  S3b = scat64(g3b, src, dst, zeros64)

  B = 2000
  x_emb = pl.pallas_call(
      _tc_d,
      grid=(N_NODES // B,),
      in_specs=[
          pl.BlockSpec((2, B, 64), lambda i: (0, i, 0)),
          pl.BlockSpec((2, B, 64), lambda i: (0, i, 0)),
          pl.BlockSpec((B, 64), lambda i: (i, 0)),
          pl.BlockSpec((B, 64), lambda i: (i, 0)),
          pl.BlockSpec((2, B, 8), lambda i: (0, i, 0)),
          pl.BlockSpec((128, 256), lambda i: (0, 0)),
          pl.BlockSpec((1, 256), lambda i: (0, 0)),
      ],
      out_specs=pl.BlockSpec((B, 256), lambda i: (i, 0)),
      out_shape=jax.ShapeDtypeStruct((N_NODES, 256), f32),
  )(S3a, S3b, g3a, g3b, deg, W3, b3.reshape(1, 256))

  PB = 2000
  x_rep = pl.pallas_call(
      _tc_pool(N_NODES // PB),
      grid=(N_NODES // PB,),
      in_specs=[
          pl.BlockSpec((PB, 1), lambda i: (i, 0)),
          pl.BlockSpec((PB, 128), lambda i: (i, 0)),
      ],
      out_specs=pl.BlockSpec((N_GRAPHS, 128), lambda i: (0, 0)),
      out_shape=jax.ShapeDtypeStruct((N_GRAPHS, 128), f32),
      scratch_shapes=[pltpu.VMEM((N_GRAPHS, 1), f32)],
  )(binds.astype(jnp.int32).reshape(N_NODES, 1), h2)

  return (x_rep, x_emb)
